# baseline (device time: 22996 ns/iter reference)
import jax
import jax.numpy as jnp
from jax import lax
from jax.experimental import pallas as pl
from jax.experimental.pallas import tpu as pltpu

N_DEV = 16
B, SQ, SKV = 2, 128, 128
H_LOC, DH = 4, 64
CHUNK = H_LOC * DH
ROWS = B * SQ
D_OUT = 512
PIECE = ROWS // N_DEV


def _body(x_ref, wq_ref, k_ref, v_ref, wo_ref, out_ref,
          stage, rs_buf, ag_buf,
          rs_send_sems, rs_recv_sems, ag_send_sems, ag_recv_sems):
    my = lax.axis_index("i")

    barrier = pltpu.get_barrier_semaphore()
    for k in range(1, N_DEV):
        pl.semaphore_signal(barrier, inc=1,
                            device_id=(lax.rem(my + k, N_DEV),),
                            device_id_type=pl.DeviceIdType.MESH)

    qi = jax.lax.broadcasted_iota(jnp.int32, (SQ, SKV), 0)
    kj = jax.lax.broadcasted_iota(jnp.int32, (SQ, SKV), 1)
    qb_, kb_ = qi // 64, kj // 64
    mask = (qb_ == kb_) | ((kb_ % 4) == (qb_ % 4))

    for b in range(B):
        xb = x_ref[b * SQ:(b + 1) * SQ, :].astype(jnp.bfloat16)
        pb = jnp.zeros((SQ, D_OUT), jnp.float32)
        for h in range(H_LOC):
            wqh = wq_ref[h].astype(jnp.bfloat16)
            kbh = k_ref[b, h].astype(jnp.bfloat16)
            vbh = v_ref[b, h].astype(jnp.bfloat16)
            woh = wo_ref[h].astype(jnp.bfloat16)
            q = jax.lax.dot_general(xb, wqh, (((1,), (0,)), ((), ())),
                                    preferred_element_type=jnp.float32)
            s = jax.lax.dot_general(q.astype(jnp.bfloat16), kbh,
                                    (((1,), (1,)), ((), ())),
                                    preferred_element_type=jnp.float32)
            e = jnp.exp(jnp.where(mask, s * 0.125, -30.0))
            w = (e / jnp.sum(e, axis=1, keepdims=True)).astype(jnp.bfloat16)
            c = jax.lax.dot_general(w, vbh, (((1,), (0,)), ((), ())),
                                    preferred_element_type=jnp.float32)
            pb = pb + jax.lax.dot_general(c.astype(jnp.bfloat16), woh,
                                          (((1,), (0,)), ((), ())),
                                          preferred_element_type=jnp.float32)
        stage[b * 8:(b + 1) * 8] = pb.astype(jnp.bfloat16).reshape(
            8, PIECE, D_OUT)

    pl.semaphore_wait(barrier, N_DEV - 1)

    rs_rdmas = []
    for k in range(1, N_DEV):
        j = lax.rem(my + k, N_DEV)
        rdma = pltpu.make_async_remote_copy(
            src_ref=stage.at[j],
            dst_ref=rs_buf.at[my],
            send_sem=rs_send_sems.at[j],
            recv_sem=rs_recv_sems.at[my],
            device_id=(j,),
            device_id_type=pl.DeviceIdType.MESH,
        )
        rdma.start()
        rs_rdmas.append(rdma)
    piece = stage[pl.ds(my, 1)][0].astype(jnp.float32)
    for k in range(1, N_DEV):
        s = lax.rem(my + k, N_DEV)
        recv = pltpu.make_async_remote_copy(
            src_ref=stage.at[s],
            dst_ref=rs_buf.at[s],
            send_sem=rs_send_sems.at[s],
            recv_sem=rs_recv_sems.at[s],
            device_id=(s,),
            device_id_type=pl.DeviceIdType.MESH,
        )
        recv.wait_recv()
        piece = piece + rs_buf[pl.ds(s, 1)][0].astype(jnp.float32)

    ag_buf[pl.ds(my, 1)] = piece.astype(jnp.bfloat16)[None]
    ag_rdmas = []
    for k in range(1, N_DEV):
        j = lax.rem(my + k, N_DEV)
        rdma = pltpu.make_async_remote_copy(
            src_ref=ag_buf.at[my],
            dst_ref=ag_buf.at[my],
            send_sem=ag_send_sems.at[j],
            recv_sem=ag_recv_sems.at[my],
            device_id=(j,),
            device_id_type=pl.DeviceIdType.MESH,
        )
        rdma.start()
        ag_rdmas.append(rdma)
    out_ref[pl.ds(my * PIECE, PIECE)] = piece
    for k in range(1, N_DEV):
        s = lax.rem(my + k, N_DEV)
        recv = pltpu.make_async_remote_copy(
            src_ref=ag_buf.at[s],
            dst_ref=ag_buf.at[s],
            send_sem=ag_send_sems.at[s],
            recv_sem=ag_recv_sems.at[s],
            device_id=(s,),
            device_id_type=pl.DeviceIdType.MESH,
        )
        recv.wait_recv()
        out_ref[pl.ds(s * PIECE, PIECE)] = (
            ag_buf[pl.ds(s, 1)][0].astype(jnp.float32))

    for rdma in rs_rdmas + ag_rdmas:
        rdma.wait_send()


def kernel(x, Wq, K_ext, V_ext, Wo):
    my = lax.axis_index("i")

    x2d = x.reshape(ROWS, x.shape[2])
    Wq_loc = lax.dynamic_slice(Wq, (0, my * CHUNK), (Wq.shape[0], CHUNK))
    wq4 = Wq_loc.reshape(Wq.shape[0], H_LOC, DH).transpose(1, 0, 2)
    Wo_loc = lax.dynamic_slice(Wo, (my * CHUNK, 0), (CHUNK, Wo.shape[1]))
    wo4 = Wo_loc.reshape(H_LOC, DH, Wo.shape[1])
    k4 = K_ext.transpose(0, 2, 1, 3)
    v4 = V_ext.transpose(0, 2, 1, 3)

    out = pl.pallas_call(
        _body,
        out_shape=jax.ShapeDtypeStruct((ROWS, D_OUT), jnp.float32),
        in_specs=[pl.BlockSpec(memory_space=pltpu.VMEM)] * 5,
        out_specs=pl.BlockSpec(memory_space=pltpu.VMEM),
        scratch_shapes=[
            pltpu.VMEM((N_DEV, PIECE, D_OUT), jnp.bfloat16),
            pltpu.VMEM((N_DEV, PIECE, D_OUT), jnp.bfloat16),
            pltpu.VMEM((N_DEV, PIECE, D_OUT), jnp.bfloat16),
            pltpu.SemaphoreType.DMA((N_DEV,)),
            pltpu.SemaphoreType.DMA((N_DEV,)),
            pltpu.SemaphoreType.DMA((N_DEV,)),
            pltpu.SemaphoreType.DMA((N_DEV,)),
        ],
        compiler_params=pltpu.CompilerParams(collective_id=0),
    )(x2d, wq4, k4, v4, wo4)
    return out.reshape(B, SQ, D_OUT)
